# baseline (speedup 1.0000x reference)
import jax
import jax.numpy as jnp
from jax import lax
from jax.experimental import pallas as pl
from jax.experimental.pallas import tpu as pltpu
from jax.experimental.pallas import tpu_sc as plsc

VOCAB = 1000000
EMBED_DIM = 64
BATCH = 16384

QUART = 262144
NWORDS = EMBED_DIM // 2
CBLK = 8192
QBLK = QUART // CBLK
LAST_CBLK = (VOCAB - 1) // CBLK

NUM_CORES = 2
NUM_SUBCORES = 16
LANES = 16

NW = NUM_CORES * NUM_SUBCORES
B_PER_W = BATCH // NW
CHUNK = 128
N_CHUNKS = B_PER_W // CHUNK


def _tc_pack_body(t0, t1, t2, t3, c0, c1, c2, c3, in2_ref, out2_ref):
    def pack(x):
        y = x[...].T
        lo = y[:, 0:NWORDS].astype(jnp.bfloat16).astype(jnp.float32)
        hi = y[:, NWORDS:2 * NWORDS].astype(jnp.bfloat16).astype(jnp.float32)
        lo_bits = lax.bitcast_convert_type(lo, jnp.int32)
        hi_bits = lax.bitcast_convert_type(hi, jnp.int32)
        return (hi_bits & jnp.int32(-65536)) | lax.shift_right_logical(
            lo_bits, 16)

    for q, blk in enumerate((t0, t1, t2, t3)):
        in2_ref[:, q * NWORDS:(q + 1) * NWORDS] = pack(blk)
    for q, blk in enumerate((c0, c1, c2, c3)):
        out2_ref[:, q * NWORDS:(q + 1) * NWORDS] = pack(blk)


def _compact_tables(tin, tout):
    def quarter(q):
        return pl.BlockSpec(
            (EMBED_DIM, CBLK),
            lambda j, q=q: (0, jnp.minimum(j + q * QBLK, LAST_CBLK)))

    out_spec = pl.BlockSpec((CBLK, 4 * NWORDS), lambda j: (j, 0))
    return pl.pallas_call(
        _tc_pack_body,
        grid=(QBLK,),
        in_specs=[quarter(q) for q in range(4)] * 2,
        out_specs=[out_spec, out_spec],
        out_shape=[jax.ShapeDtypeStruct((QUART, 4 * NWORDS), jnp.int32)] * 2,
    )(tin, tin, tin, tin, tout, tout, tout, tout)


def _sc_body(tgt_idx_hbm, ctx_idx_hbm, in2_hbm, out2_hbm, score_hbm,
             tgt_idx_v, ctx_idx_v, tgt_row_v, ctx_row_v,
             tgt_rows_a, ctx_rows_a, tgt_rows_b, ctx_rows_b, score_v,
             sem_ta, sem_ca, sem_tb, sem_cb):
    wid = lax.axis_index("s") * NUM_CORES + lax.axis_index("c")
    base = wid * B_PER_W

    pltpu.sync_copy(tgt_idx_hbm.at[pl.ds(base, B_PER_W)], tgt_idx_v)
    pltpu.sync_copy(ctx_idx_hbm.at[pl.ds(base, B_PER_W)], ctx_idx_v)

    def rowidx(g, c):
        s = pl.ds(g * LANES, LANES)
        tgt_row_v[s] = tgt_idx_v[s] & (QUART - 1)
        ctx_row_v[s] = ctx_idx_v[s] & (QUART - 1)
        return c

    lax.fori_loop(0, B_PER_W // LANES, rowidx, 0)

    lane_iota = lax.iota(jnp.int32, LANES)
    bufs = [(tgt_rows_a, ctx_rows_a, sem_ta, sem_ca),
            (tgt_rows_b, ctx_rows_b, sem_tb, sem_cb)]

    def issue(ck):
        trows, crows, st, sc = bufs[ck % 2]
        row0 = ck * CHUNK
        cp_t = pltpu.async_copy(
            in2_hbm.at[tgt_row_v.at[pl.ds(row0, CHUNK)]], trows, st)
        cp_c = pltpu.async_copy(
            out2_hbm.at[ctx_row_v.at[pl.ds(row0, CHUNK)]], crows, sc)
        return cp_t, cp_c

    def compute(ck):
        trows, crows, _, _ = bufs[ck % 2]
        row0 = ck * CHUNK

        def group(g, c2):
            s = pl.ds(row0 + g * LANES, LANES)
            rows = g * LANES + lane_iota
            tcol = (tgt_idx_v[s] >> 18) * NWORDS
            ccol = (ctx_idx_v[s] >> 18) * NWORDS
            acc = jnp.zeros((LANES,), jnp.float32)
            for d in range(NWORDS):
                tw = plsc.load_gather(trows, [rows, tcol + d])
                cw = plsc.load_gather(crows, [rows, ccol + d])
                ta, tb = plsc.unpack(plsc.bitcast(tw, jnp.bfloat16),
                                     format=plsc.PackFormat.INTERLEAVED)
                ca, cb = plsc.unpack(plsc.bitcast(cw, jnp.bfloat16),
                                     format=plsc.PackFormat.INTERLEAVED)
                acc = acc + ta * ca + tb * cb
            score_v[s] = acc
            return c2

        lax.fori_loop(0, CHUNK // LANES, group, 0)

    pending = [None] * N_CHUNKS
    for ck in range(N_CHUNKS):
        pending[ck] = issue(ck)
        if ck >= 1:
            for cp in pending[ck - 1]:
                cp.wait()
            compute(ck - 1)
    for cp in pending[N_CHUNKS - 1]:
        cp.wait()
    compute(N_CHUNKS - 1)

    pltpu.sync_copy(score_v, score_hbm.at[pl.ds(base, B_PER_W)])


@jax.jit
def _w2v_scores(tgt_idx, ctx_idx, in_emb, out_emb):
    in2, out2 = _compact_tables(in_emb.T, out_emb.T)
    mesh = plsc.VectorSubcoreMesh(
        core_axis_name="c", subcore_axis_name="s",
        num_cores=NUM_CORES, num_subcores=NUM_SUBCORES)
    return pl.kernel(
        _sc_body,
        out_type=jax.ShapeDtypeStruct((BATCH,), jnp.float32),
        mesh=mesh,
        scratch_types=[
            pltpu.VMEM((B_PER_W,), jnp.int32),
            pltpu.VMEM((B_PER_W,), jnp.int32),
            pltpu.VMEM((B_PER_W,), jnp.int32),
            pltpu.VMEM((B_PER_W,), jnp.int32),
            pltpu.VMEM((CHUNK, 4 * NWORDS), jnp.int32),
            pltpu.VMEM((CHUNK, 4 * NWORDS), jnp.int32),
            pltpu.VMEM((CHUNK, 4 * NWORDS), jnp.int32),
            pltpu.VMEM((CHUNK, 4 * NWORDS), jnp.int32),
            pltpu.VMEM((B_PER_W,), jnp.float32),
            pltpu.SemaphoreType.DMA,
            pltpu.SemaphoreType.DMA,
            pltpu.SemaphoreType.DMA,
            pltpu.SemaphoreType.DMA,
        ],
        compiler_params=pltpu.CompilerParams(needs_layout_passes=False),
    )(tgt_idx, ctx_idx, in2, out2)


def kernel(target_word_idx, context_word_idx, in_emb, out_emb):
    tgt = target_word_idx.astype(jnp.int32)
    ctx = context_word_idx.astype(jnp.int32)
    return _w2v_scores(tgt, ctx, in_emb, out_emb)


# speedup vs baseline: 1.7479x; 1.7479x over previous
import jax
import jax.numpy as jnp
from jax import lax
from jax.experimental import pallas as pl
from jax.experimental.pallas import tpu as pltpu
from jax.experimental.pallas import tpu_sc as plsc

VOCAB = 1000000
EMBED_DIM = 64
BATCH = 16384

HALF = 524288
CBLK = 8192
RBLK = HALF // CBLK
LAST_CBLK = (VOCAB - 1) // CBLK

NUM_CORES = 2
NUM_SUBCORES = 16
LANES = 16

NW = NUM_CORES * NUM_SUBCORES
B_PER_W = BATCH // NW
CHUNK = 128
N_CHUNKS = B_PER_W // CHUNK


def _tc_transpose_body(ta, tb, ca, cb, in2_ref, out2_ref):
    in2_ref[:, 0:EMBED_DIM] = ta[...].astype(jnp.bfloat16).T.astype(jnp.float32)
    in2_ref[:, EMBED_DIM:2 * EMBED_DIM] = tb[...].astype(jnp.bfloat16).T.astype(jnp.float32)
    out2_ref[:, 0:EMBED_DIM] = ca[...].astype(jnp.bfloat16).T.astype(jnp.float32)
    out2_ref[:, EMBED_DIM:2 * EMBED_DIM] = cb[...].astype(jnp.bfloat16).T.astype(jnp.float32)


def _compact_tables(tin, tout):
    lo = pl.BlockSpec((EMBED_DIM, CBLK), lambda j: (0, j))
    hi = pl.BlockSpec((EMBED_DIM, CBLK),
                      lambda j: (0, jnp.minimum(j + RBLK, LAST_CBLK)))
    out_spec = pl.BlockSpec((CBLK, 2 * EMBED_DIM), lambda j: (j, 0))
    return pl.pallas_call(
        _tc_transpose_body,
        grid=(RBLK,),
        in_specs=[lo, hi, lo, hi],
        out_specs=[out_spec, out_spec],
        out_shape=[jax.ShapeDtypeStruct((HALF, 2 * EMBED_DIM), jnp.float32)] * 2,
        compiler_params=pltpu.CompilerParams(fuse_transposed_lhs_in_matmul=True),
    )(tin, tin, tout, tout)


def _sc_body(tgt_idx_hbm, ctx_idx_hbm, in2_hbm, out2_hbm, score_hbm,
             tgt_idx_v, ctx_idx_v, tgt_row_v, ctx_row_v,
             tgt_rows_a, ctx_rows_a, tgt_rows_b, ctx_rows_b, score_v,
             sem_ta, sem_ca, sem_tb, sem_cb):
    wid = lax.axis_index("s") * NUM_CORES + lax.axis_index("c")
    base = wid * B_PER_W

    pltpu.sync_copy(tgt_idx_hbm.at[pl.ds(base, B_PER_W)], tgt_idx_v)
    pltpu.sync_copy(ctx_idx_hbm.at[pl.ds(base, B_PER_W)], ctx_idx_v)

    def rowidx(g, c):
        s = pl.ds(g * LANES, LANES)
        tgt_row_v[s] = tgt_idx_v[s] & (HALF - 1)
        ctx_row_v[s] = ctx_idx_v[s] & (HALF - 1)
        return c

    lax.fori_loop(0, B_PER_W // LANES, rowidx, 0)

    lane_iota = lax.iota(jnp.int32, LANES)
    bufs = [(tgt_rows_a, ctx_rows_a, sem_ta, sem_ca),
            (tgt_rows_b, ctx_rows_b, sem_tb, sem_cb)]

    def issue(ck):
        trows, crows, st, sc = bufs[ck % 2]
        row0 = ck * CHUNK
        cp_t = pltpu.async_copy(
            in2_hbm.at[tgt_row_v.at[pl.ds(row0, CHUNK)]], trows, st)
        cp_c = pltpu.async_copy(
            out2_hbm.at[ctx_row_v.at[pl.ds(row0, CHUNK)]], crows, sc)
        return cp_t, cp_c

    def compute(ck):
        trows, crows, _, _ = bufs[ck % 2]
        row0 = ck * CHUNK

        def group(g, c2):
            s = pl.ds(row0 + g * LANES, LANES)
            rows = g * LANES + lane_iota
            tcol = (tgt_idx_v[s] >> 19) * EMBED_DIM
            ccol = (ctx_idx_v[s] >> 19) * EMBED_DIM
            acc = jnp.zeros((LANES,), jnp.float32)
            for d in range(EMBED_DIM):
                tv = plsc.load_gather(trows, [rows, tcol + d])
                cv = plsc.load_gather(crows, [rows, ccol + d])
                acc = acc + tv * cv
            score_v[s] = acc
            return c2

        lax.fori_loop(0, CHUNK // LANES, group, 0)

    pending = [None] * N_CHUNKS
    for ck in range(N_CHUNKS):
        pending[ck] = issue(ck)
        if ck >= 1:
            for cp in pending[ck - 1]:
                cp.wait()
            compute(ck - 1)
    for cp in pending[N_CHUNKS - 1]:
        cp.wait()
    compute(N_CHUNKS - 1)

    pltpu.sync_copy(score_v, score_hbm.at[pl.ds(base, B_PER_W)])


@jax.jit
def _w2v_scores(tgt_idx, ctx_idx, in_emb, out_emb):
    in2, out2 = _compact_tables(in_emb.T, out_emb.T)
    mesh = plsc.VectorSubcoreMesh(
        core_axis_name="c", subcore_axis_name="s",
        num_cores=NUM_CORES, num_subcores=NUM_SUBCORES)
    return pl.kernel(
        _sc_body,
        out_type=jax.ShapeDtypeStruct((BATCH,), jnp.float32),
        mesh=mesh,
        scratch_types=[
            pltpu.VMEM((B_PER_W,), jnp.int32),
            pltpu.VMEM((B_PER_W,), jnp.int32),
            pltpu.VMEM((B_PER_W,), jnp.int32),
            pltpu.VMEM((B_PER_W,), jnp.int32),
            pltpu.VMEM((CHUNK, 2 * EMBED_DIM), jnp.float32),
            pltpu.VMEM((CHUNK, 2 * EMBED_DIM), jnp.float32),
            pltpu.VMEM((CHUNK, 2 * EMBED_DIM), jnp.float32),
            pltpu.VMEM((CHUNK, 2 * EMBED_DIM), jnp.float32),
            pltpu.VMEM((B_PER_W,), jnp.float32),
            pltpu.SemaphoreType.DMA,
            pltpu.SemaphoreType.DMA,
            pltpu.SemaphoreType.DMA,
            pltpu.SemaphoreType.DMA,
        ],
        compiler_params=pltpu.CompilerParams(needs_layout_passes=False),
    )(tgt_idx, ctx_idx, in2, out2)


def kernel(target_word_idx, context_word_idx, in_emb, out_emb):
    tgt = target_word_idx.astype(jnp.int32)
    ctx = context_word_idx.astype(jnp.int32)
    return _w2v_scores(tgt, ctx, in_emb, out_emb)


# speedup vs baseline: 1.7533x; 1.0031x over previous
import jax
import jax.numpy as jnp
from jax import lax
from jax.experimental import pallas as pl
from jax.experimental.pallas import tpu as pltpu
from jax.experimental.pallas import tpu_sc as plsc

VOCAB = 1000000
EMBED_DIM = 64
BATCH = 16384

HALF = 524288
CBLK = 8192
RBLK = HALF // CBLK
LAST_CBLK = (VOCAB - 1) // CBLK

NUM_CORES = 2
NUM_SUBCORES = 16
LANES = 16

NW = NUM_CORES * NUM_SUBCORES
B_PER_W = BATCH // NW
CHUNK = 128
N_CHUNKS = B_PER_W // CHUNK


def _tc_transpose_body(ta, tb, ca, cb, in2_ref, out2_ref):
    in2_ref[:, 0:EMBED_DIM] = ta[...].astype(jnp.bfloat16).T.astype(jnp.float32)
    in2_ref[:, EMBED_DIM:2 * EMBED_DIM] = tb[...].astype(jnp.bfloat16).T.astype(jnp.float32)
    out2_ref[:, 0:EMBED_DIM] = ca[...].astype(jnp.bfloat16).T.astype(jnp.float32)
    out2_ref[:, EMBED_DIM:2 * EMBED_DIM] = cb[...].astype(jnp.bfloat16).T.astype(jnp.float32)


def _compact_tables(tin, tout):
    lo = pl.BlockSpec((EMBED_DIM, CBLK), lambda j: (0, j))
    hi = pl.BlockSpec((EMBED_DIM, CBLK),
                      lambda j: (0, jnp.minimum(j + RBLK, LAST_CBLK)))
    out_spec = pl.BlockSpec((CBLK, 2 * EMBED_DIM), lambda j: (j, 0))
    return pl.pallas_call(
        _tc_transpose_body,
        grid=(RBLK,),
        in_specs=[lo, hi, lo, hi],
        out_specs=[out_spec, out_spec],
        out_shape=[jax.ShapeDtypeStruct((HALF, 2 * EMBED_DIM), jnp.float32)] * 2,
    )(tin, tin, tout, tout)


def _sc_body(tgt_idx_hbm, ctx_idx_hbm, in2_hbm, out2_hbm, score_hbm,
             tgt_idx_v, ctx_idx_v, tgt_row_v, ctx_row_v,
             tgt_rows_a, ctx_rows_a, tgt_rows_b, ctx_rows_b, score_v,
             sem_ta, sem_ca, sem_tb, sem_cb):
    wid = lax.axis_index("s") * NUM_CORES + lax.axis_index("c")
    base = wid * B_PER_W

    pltpu.sync_copy(tgt_idx_hbm.at[pl.ds(base, B_PER_W)], tgt_idx_v)
    pltpu.sync_copy(ctx_idx_hbm.at[pl.ds(base, B_PER_W)], ctx_idx_v)

    def rowidx(g, c):
        s = pl.ds(g * LANES, LANES)
        tgt_row_v[s] = tgt_idx_v[s] & (HALF - 1)
        ctx_row_v[s] = ctx_idx_v[s] & (HALF - 1)
        return c

    lax.fori_loop(0, B_PER_W // LANES, rowidx, 0)

    lane_iota = lax.iota(jnp.int32, LANES)
    bufs = [(tgt_rows_a, ctx_rows_a, sem_ta, sem_ca),
            (tgt_rows_b, ctx_rows_b, sem_tb, sem_cb)]

    def issue(ck):
        trows, crows, st, sc = bufs[ck % 2]
        row0 = ck * CHUNK
        cp_t = pltpu.async_copy(
            in2_hbm.at[tgt_row_v.at[pl.ds(row0, CHUNK)]], trows, st)
        cp_c = pltpu.async_copy(
            out2_hbm.at[ctx_row_v.at[pl.ds(row0, CHUNK)]], crows, sc)
        return cp_t, cp_c

    def compute(ck):
        trows, crows, _, _ = bufs[ck % 2]
        row0 = ck * CHUNK

        def group(g, c2):
            s = pl.ds(row0 + g * LANES, LANES)
            rows = g * LANES + lane_iota
            tcol = (tgt_idx_v[s] >> 19) * EMBED_DIM
            ccol = (ctx_idx_v[s] >> 19) * EMBED_DIM
            acc = jnp.zeros((LANES,), jnp.float32)
            for d in range(EMBED_DIM):
                tv = plsc.load_gather(trows, [rows, tcol + d])
                cv = plsc.load_gather(crows, [rows, ccol + d])
                acc = acc + tv * cv
            score_v[s] = acc
            return c2

        lax.fori_loop(0, CHUNK // LANES, group, 0)

    pending = [None] * N_CHUNKS
    for ck in range(N_CHUNKS):
        pending[ck] = issue(ck)
        if ck >= 1:
            for cp in pending[ck - 1]:
                cp.wait()
            compute(ck - 1)
    for cp in pending[N_CHUNKS - 1]:
        cp.wait()
    compute(N_CHUNKS - 1)

    pltpu.sync_copy(score_v, score_hbm.at[pl.ds(base, B_PER_W)])


@jax.jit
def _w2v_scores(tgt_idx, ctx_idx, in_emb, out_emb):
    in2, out2 = _compact_tables(in_emb.T, out_emb.T)
    mesh = plsc.VectorSubcoreMesh(
        core_axis_name="c", subcore_axis_name="s",
        num_cores=NUM_CORES, num_subcores=NUM_SUBCORES)
    return pl.kernel(
        _sc_body,
        out_type=jax.ShapeDtypeStruct((BATCH,), jnp.float32),
        mesh=mesh,
        scratch_types=[
            pltpu.VMEM((B_PER_W,), jnp.int32),
            pltpu.VMEM((B_PER_W,), jnp.int32),
            pltpu.VMEM((B_PER_W,), jnp.int32),
            pltpu.VMEM((B_PER_W,), jnp.int32),
            pltpu.VMEM((CHUNK, 2 * EMBED_DIM), jnp.float32),
            pltpu.VMEM((CHUNK, 2 * EMBED_DIM), jnp.float32),
            pltpu.VMEM((CHUNK, 2 * EMBED_DIM), jnp.float32),
            pltpu.VMEM((CHUNK, 2 * EMBED_DIM), jnp.float32),
            pltpu.VMEM((B_PER_W,), jnp.float32),
            pltpu.SemaphoreType.DMA,
            pltpu.SemaphoreType.DMA,
            pltpu.SemaphoreType.DMA,
            pltpu.SemaphoreType.DMA,
        ],
        compiler_params=pltpu.CompilerParams(needs_layout_passes=False),
    )(tgt_idx, ctx_idx, in2, out2)


def kernel(target_word_idx, context_word_idx, in_emb, out_emb):
    tgt = target_word_idx.astype(jnp.int32)
    ctx = context_word_idx.astype(jnp.int32)
    return _w2v_scores(tgt, ctx, in_emb, out_emb)
